# baseline (speedup 1.0000x reference)
import jax
import jax.numpy as jnp
from jax.experimental import pallas as pl

_PERIODS = (4, 2, 1)
_TOPM = 20
_TEMP = 0.1
_EPS = 1e-12


def _decompose_norm(data):
    N, S, C = data.shape
    outs = []
    for g in _PERIODS:
        if g == 1:
            cur = data
        else:
            bm = data.reshape(N, S // g, g, C).mean(axis=2)
            rep = jnp.broadcast_to(bm[:, :, None, :], (N, S // g, g, C))
            cur = rep.reshape(N, S, C)
        cur = cur - cur[:, -1:, :]
        cur = cur - cur.mean(axis=1, keepdims=True)
        n = jnp.sqrt(jnp.sum(cur * cur, axis=1, keepdims=True))
        cur = cur / jnp.maximum(n, _EPS)
        outs.append(cur.transpose(2, 0, 1))
    return outs


def _sim_kernel(x_ref, tr_ref, out_ref):
    qs = _decompose_norm(x_ref[...])
    ts = _decompose_norm(tr_ref[...])
    acc = None
    for q, t in zip(qs, ts):
        p = jax.lax.dot_general(
            t, q,
            dimension_numbers=(((2,), (2,)), ((0,), (0,))),
            preferred_element_type=jnp.float32)
        acc = p if acc is None else acc + p
    out_ref[...] = acc * (1.0 / len(_PERIODS))


def _ybank_kernel(y_ref, out_ref):
    y = y_ref[...]
    N, P, C = y.shape
    acc = None
    for g in _PERIODS:
        if g == 1:
            bm = y
        else:
            bm = y.reshape(N, P // g, g, C).mean(axis=2)
        rep = jnp.broadcast_to(bm[:, :, None, :], (N, P // g, g, C))
        cur = rep.reshape(N, P, C)
        cur = cur - cur[:, -1:, :]
        acc = cur if acc is None else acc + cur
    out_ref[...] = acc * (1.0 / len(_PERIODS))


def _retrieve_kernel(sim_ref, yb_ref, out_ref):
    s = sim_ref[...]
    work = s
    for _ in range(_TOPM - 1):
        m = jnp.max(work, axis=1, keepdims=True)
        work = jnp.where(work >= m, -jnp.inf, work)
    thresh = jnp.max(work, axis=1, keepdims=True)
    m1 = jnp.max(s, axis=1, keepdims=True)
    e = jnp.where(s >= thresh, jnp.exp((s - m1) / _TEMP), 0.0)
    w = e / jnp.sum(e, axis=1, keepdims=True)
    out_ref[...] = jax.lax.dot_general(
        w, yb_ref[...],
        dimension_numbers=(((1,), (1,)), ((0,), (0,))),
        precision=jax.lax.Precision.HIGHEST,
        preferred_element_type=jnp.float32)


def kernel(x, train_data, y_data):
    B, S, C = x.shape
    T = train_data.shape[0]
    P = y_data.shape[1]

    TB = 16
    sim = pl.pallas_call(
        _sim_kernel,
        grid=(T // TB,),
        in_specs=[
            pl.BlockSpec((B, S, C), lambda i: (0, 0, 0)),
            pl.BlockSpec((TB, S, C), lambda i: (i, 0, 0)),
        ],
        out_specs=pl.BlockSpec((C, TB, B), lambda i: (0, i, 0)),
        out_shape=jax.ShapeDtypeStruct((C, T, B), jnp.float32),
    )(x, train_data)

    YB = 128
    y_bank = pl.pallas_call(
        _ybank_kernel,
        grid=(T // YB,),
        in_specs=[pl.BlockSpec((YB, P, C), lambda i: (i, 0, 0))],
        out_specs=pl.BlockSpec((YB, P, C), lambda i: (i, 0, 0)),
        out_shape=jax.ShapeDtypeStruct((T, P, C), jnp.float32),
    )(y_data)
    y_bank_t = y_bank.transpose(2, 0, 1)

    CB = 8
    out_cbp = pl.pallas_call(
        _retrieve_kernel,
        grid=(C // CB,),
        in_specs=[
            pl.BlockSpec((CB, T, B), lambda i: (i, 0, 0)),
            pl.BlockSpec((CB, T, P), lambda i: (i, 0, 0)),
        ],
        out_specs=pl.BlockSpec((CB, B, P), lambda i: (i, 0, 0)),
        out_shape=jax.ShapeDtypeStruct((C, B, P), jnp.float32),
    )(sim, y_bank_t)
    return out_cbp.transpose(1, 2, 0)


# speedup vs baseline: 1.9021x; 1.9021x over previous
import jax
import jax.numpy as jnp
from jax.experimental import pallas as pl

_PERIODS = (4, 2, 1)
_TOPM = 20
_TEMP = 0.1
_EPS = 1e-12


def _decompose_norm(data):
    N, S, C = data.shape
    outs = []
    for g in _PERIODS:
        if g == 1:
            cur = data
        else:
            bm = data.reshape(N, S // g, g, C).mean(axis=2)
            rep = jnp.broadcast_to(bm[:, :, None, :], (N, S // g, g, C))
            cur = rep.reshape(N, S, C)
        cur = cur - cur[:, -1:, :]
        cur = cur - cur.mean(axis=1, keepdims=True)
        n = jnp.sqrt(jnp.sum(cur * cur, axis=1, keepdims=True))
        cur = cur / jnp.maximum(n, _EPS)
        outs.append(cur.transpose(2, 0, 1))
    return outs


def _query_kernel(x_ref, out_ref):
    out_ref[...] = jnp.stack(_decompose_norm(x_ref[...]), axis=0)


def _sim_kernel(q_ref, tr_ref, out_ref):
    qs = [q_ref[g] for g in range(len(_PERIODS))]
    ts = _decompose_norm(tr_ref[...])
    acc = None
    for q, t in zip(qs, ts):
        p = jax.lax.dot_general(
            t, q,
            dimension_numbers=(((2,), (2,)), ((0,), (0,))),
            preferred_element_type=jnp.float32)
        acc = p if acc is None else acc + p
    out_ref[...] = acc * (1.0 / len(_PERIODS))


def _ybank_lanes(y):
    cb, t, P = y.shape
    lane = jax.lax.broadcasted_iota(jnp.int32, (cb, t, P), 2)
    r2 = (y + jnp.where(lane % 2 == 0,
                        jnp.roll(y, -1, axis=2),
                        jnp.roll(y, 1, axis=2))) * 0.5
    r4 = (r2 + jnp.where(lane % 4 < 2,
                         jnp.roll(r2, -2, axis=2),
                         jnp.roll(r2, 2, axis=2))) * 0.5
    acc = None
    for cur in (r4, r2, y):
        cur = cur - cur[:, :, P - 1:P]
        acc = cur if acc is None else acc + cur
    return acc * (1.0 / len(_PERIODS))


def _retrieve_kernel(sim_ref, yt_ref, out_ref):
    s = sim_ref[...]
    work = s
    for _ in range(_TOPM - 1):
        m = jnp.max(work, axis=-1, keepdims=True)
        work = jnp.where(work >= m, -jnp.inf, work)
    thresh = jnp.max(work, axis=-1, keepdims=True)
    m1 = jnp.max(s, axis=-1, keepdims=True)
    e = jnp.where(s >= thresh, jnp.exp((s - m1) / _TEMP), 0.0)
    w = e / jnp.sum(e, axis=-1, keepdims=True)
    bank = _ybank_lanes(yt_ref[...])
    out_ref[...] = jax.lax.dot_general(
        w, bank,
        dimension_numbers=(((2,), (1,)), ((0,), (0,))),
        precision=jax.lax.Precision.HIGHEST,
        preferred_element_type=jnp.float32)


def kernel(x, train_data, y_data):
    B, S, C = x.shape
    G = len(_PERIODS)
    T = train_data.shape[0]
    P = y_data.shape[1]

    qs = pl.pallas_call(
        _query_kernel,
        out_shape=jax.ShapeDtypeStruct((G, C, B, S), jnp.float32),
    )(x)

    TB = 32
    sim = pl.pallas_call(
        _sim_kernel,
        grid=(T // TB,),
        in_specs=[
            pl.BlockSpec((G, C, B, S), lambda i: (0, 0, 0, 0)),
            pl.BlockSpec((TB, S, C), lambda i: (i, 0, 0)),
        ],
        out_specs=pl.BlockSpec((C, TB, B), lambda i: (0, i, 0)),
        out_shape=jax.ShapeDtypeStruct((C, T, B), jnp.float32),
    )(qs, train_data)
    sim_t = sim.transpose(0, 2, 1)
    y_t = y_data.transpose(2, 0, 1)

    CB = 4
    out_cbp = pl.pallas_call(
        _retrieve_kernel,
        grid=(C // CB,),
        in_specs=[
            pl.BlockSpec((CB, B, T), lambda i: (i, 0, 0)),
            pl.BlockSpec((CB, T, P), lambda i: (i, 0, 0)),
        ],
        out_specs=pl.BlockSpec((CB, B, P), lambda i: (i, 0, 0)),
        out_shape=jax.ShapeDtypeStruct((C, B, P), jnp.float32),
    )(sim_t, y_t)
    return out_cbp.transpose(1, 2, 0)


# speedup vs baseline: 7.0027x; 3.6815x over previous
import jax
import jax.numpy as jnp
from jax.experimental import pallas as pl

_PERIODS = (4, 2, 1)
_TOPM = 20
_TEMP = 0.1
_EPS = 1e-12


def _decompose_norm(data):
    N, S, C = data.shape
    outs = []
    for g in _PERIODS:
        if g == 1:
            cur = data
        else:
            bm = data.reshape(N, S // g, g, C).mean(axis=2)
            rep = jnp.broadcast_to(bm[:, :, None, :], (N, S // g, g, C))
            cur = rep.reshape(N, S, C)
        cur = cur - cur[:, -1:, :]
        cur = cur - cur.mean(axis=1, keepdims=True)
        n = jnp.sqrt(jnp.sum(cur * cur, axis=1, keepdims=True))
        cur = cur / jnp.maximum(n, _EPS)
        outs.append(cur.transpose(2, 0, 1))
    return outs


def _query_kernel(x_ref, out_ref):
    out_ref[...] = jnp.stack(_decompose_norm(x_ref[...]), axis=0)


def _sim_kernel(q_ref, tr_ref, out_ref):
    wt = tr_ref[...].transpose(2, 0, 1)
    cc, tb, S = wt.shape
    lane = jax.lax.broadcasted_iota(jnp.int32, (cc, tb, S), 2)
    r2 = (wt + jnp.where(lane % 2 == 0,
                         jnp.roll(wt, -1, axis=2),
                         jnp.roll(wt, 1, axis=2))) * 0.5
    r4 = (r2 + jnp.where(lane % 4 < 2,
                         jnp.roll(r2, -2, axis=2),
                         jnp.roll(r2, 2, axis=2))) * 0.5
    reps = {4: r4, 2: r2, 1: wt}
    acc = None
    for g_i, g in enumerate(_PERIODS):
        cur = reps[g]
        cur = cur - cur[:, :, S - 1:S]
        cur = cur - cur.mean(axis=2, keepdims=True)
        n = jnp.sqrt(jnp.sum(cur * cur, axis=2, keepdims=True))
        cur = cur / jnp.maximum(n, _EPS)
        p = jax.lax.dot_general(
            cur, q_ref[g_i],
            dimension_numbers=(((2,), (2,)), ((0,), (0,))),
            preferred_element_type=jnp.float32)
        acc = p if acc is None else acc + p
    out_ref[...] = acc * (1.0 / len(_PERIODS))


def _ybank_lanes(y):
    cb, t, P = y.shape
    lane = jax.lax.broadcasted_iota(jnp.int32, (cb, t, P), 2)
    r2 = (y + jnp.where(lane % 2 == 0,
                        jnp.roll(y, -1, axis=2),
                        jnp.roll(y, 1, axis=2))) * 0.5
    r4 = (r2 + jnp.where(lane % 4 < 2,
                         jnp.roll(r2, -2, axis=2),
                         jnp.roll(r2, 2, axis=2))) * 0.5
    acc = None
    for cur in (r4, r2, y):
        cur = cur - cur[:, :, P - 1:P]
        acc = cur if acc is None else acc + cur
    return acc * (1.0 / len(_PERIODS))


def _retrieve_kernel(sim_ref, yt_ref, out_ref):
    s = sim_ref[...]
    work = s
    for _ in range(_TOPM - 1):
        m = jnp.max(work, axis=-1, keepdims=True)
        work = jnp.where(work >= m, -jnp.inf, work)
    thresh = jnp.max(work, axis=-1, keepdims=True)
    m1 = jnp.max(s, axis=-1, keepdims=True)
    e = jnp.where(s >= thresh, jnp.exp((s - m1) / _TEMP), 0.0)
    w = e / jnp.sum(e, axis=-1, keepdims=True)
    bank = _ybank_lanes(yt_ref[...])
    out_ref[...] = jax.lax.dot_general(
        w, bank,
        dimension_numbers=(((2,), (1,)), ((0,), (0,))),
        precision=jax.lax.Precision.HIGHEST,
        preferred_element_type=jnp.float32)


def kernel(x, train_data, y_data):
    B, S, C = x.shape
    G = len(_PERIODS)
    T = train_data.shape[0]
    P = y_data.shape[1]

    qs = pl.pallas_call(
        _query_kernel,
        out_shape=jax.ShapeDtypeStruct((G, C, B, S), jnp.float32),
    )(x)

    TB = 32
    sim = pl.pallas_call(
        _sim_kernel,
        grid=(T // TB,),
        in_specs=[
            pl.BlockSpec((G, C, B, S), lambda i: (0, 0, 0, 0)),
            pl.BlockSpec((TB, S, C), lambda i: (i, 0, 0)),
        ],
        out_specs=pl.BlockSpec((C, TB, B), lambda i: (0, i, 0)),
        out_shape=jax.ShapeDtypeStruct((C, T, B), jnp.float32),
    )(qs, train_data)
    sim_t = sim.transpose(0, 2, 1)
    y_t = y_data.transpose(2, 0, 1)

    CB = 4
    out_cbp = pl.pallas_call(
        _retrieve_kernel,
        grid=(C // CB,),
        in_specs=[
            pl.BlockSpec((CB, B, T), lambda i: (i, 0, 0)),
            pl.BlockSpec((CB, T, P), lambda i: (i, 0, 0)),
        ],
        out_specs=pl.BlockSpec((CB, B, P), lambda i: (i, 0, 0)),
        out_shape=jax.ShapeDtypeStruct((C, B, P), jnp.float32),
    )(sim_t, y_t)
    return out_cbp.transpose(1, 2, 0)


# speedup vs baseline: 7.1446x; 1.0203x over previous
import jax
import jax.numpy as jnp
from jax.experimental import pallas as pl

_PERIODS = (4, 2, 1)
_TOPM = 20
_TEMP = 0.1
_EPS = 1e-12


def _decompose_norm(data):
    N, S, C = data.shape
    outs = []
    for g in _PERIODS:
        if g == 1:
            cur = data
        else:
            bm = data.reshape(N, S // g, g, C).mean(axis=2)
            rep = jnp.broadcast_to(bm[:, :, None, :], (N, S // g, g, C))
            cur = rep.reshape(N, S, C)
        cur = cur - cur[:, -1:, :]
        cur = cur - cur.mean(axis=1, keepdims=True)
        n = jnp.sqrt(jnp.sum(cur * cur, axis=1, keepdims=True))
        cur = cur / jnp.maximum(n, _EPS)
        outs.append(cur.transpose(2, 0, 1))
    return outs


def _query_kernel(x_ref, out_ref):
    out_ref[...] = jnp.stack(_decompose_norm(x_ref[...]), axis=0)


def _sim_kernel(q_ref, tr_ref, out_ref):
    wt = tr_ref[...].transpose(2, 0, 1)
    cc, tb, S = wt.shape
    lane = jax.lax.broadcasted_iota(jnp.int32, (cc, tb, S), 2)
    r2 = (wt + jnp.where(lane % 2 == 0,
                         jnp.roll(wt, -1, axis=2),
                         jnp.roll(wt, 1, axis=2))) * 0.5
    r4 = (r2 + jnp.where(lane % 4 < 2,
                         jnp.roll(r2, -2, axis=2),
                         jnp.roll(r2, 2, axis=2))) * 0.5
    reps = {4: r4, 2: r2, 1: wt}
    acc = None
    for g_i, g in enumerate(_PERIODS):
        cur = reps[g]
        cur = cur - cur[:, :, S - 1:S]
        cur = cur - cur.mean(axis=2, keepdims=True)
        n = jnp.sqrt(jnp.sum(cur * cur, axis=2, keepdims=True))
        cur = cur / jnp.maximum(n, _EPS)
        p = jax.lax.dot_general(
            cur, q_ref[g_i],
            dimension_numbers=(((2,), (2,)), ((0,), (0,))),
            preferred_element_type=jnp.float32)
        acc = p if acc is None else acc + p
    out_ref[...] = acc * (1.0 / len(_PERIODS))


def _ybank_lanes(y):
    cb, t, P = y.shape
    lane = jax.lax.broadcasted_iota(jnp.int32, (cb, t, P), 2)
    r2 = (y + jnp.where(lane % 2 == 0,
                        jnp.roll(y, -1, axis=2),
                        jnp.roll(y, 1, axis=2))) * 0.5
    r4 = (r2 + jnp.where(lane % 4 < 2,
                         jnp.roll(r2, -2, axis=2),
                         jnp.roll(r2, 2, axis=2))) * 0.5
    acc = None
    for cur in (r4, r2, y):
        cur = cur - cur[:, :, P - 1:P]
        acc = cur if acc is None else acc + cur
    return acc * (1.0 / len(_PERIODS))


def _retrieve_kernel(sim_ref, yt_ref, out_ref):
    s = sim_ref[...]
    work = s
    for _ in range(_TOPM - 1):
        m = jnp.max(work, axis=-1, keepdims=True)
        work = jnp.where(work >= m, -jnp.inf, work)
    thresh = jnp.max(work, axis=-1, keepdims=True)
    m1 = jnp.max(s, axis=-1, keepdims=True)
    e = jnp.where(s >= thresh, jnp.exp((s - m1) / _TEMP), 0.0)
    w = e / jnp.sum(e, axis=-1, keepdims=True)
    bank = _ybank_lanes(yt_ref[...])
    out_ref[...] = jax.lax.dot_general(
        w, bank,
        dimension_numbers=(((2,), (1,)), ((0,), (0,))),
        precision=jax.lax.Precision.HIGHEST,
        preferred_element_type=jnp.float32)


def kernel(x, train_data, y_data):
    B, S, C = x.shape
    G = len(_PERIODS)
    T = train_data.shape[0]
    P = y_data.shape[1]

    qs = pl.pallas_call(
        _query_kernel,
        out_shape=jax.ShapeDtypeStruct((G, C, B, S), jnp.float32),
    )(x)

    TB = 64
    sim = pl.pallas_call(
        _sim_kernel,
        grid=(T // TB,),
        in_specs=[
            pl.BlockSpec((G, C, B, S), lambda i: (0, 0, 0, 0)),
            pl.BlockSpec((TB, S, C), lambda i: (i, 0, 0)),
        ],
        out_specs=pl.BlockSpec((C, TB, B), lambda i: (0, i, 0)),
        out_shape=jax.ShapeDtypeStruct((C, T, B), jnp.float32),
    )(qs, train_data)
    sim_t = sim.transpose(0, 2, 1)
    y_t = y_data.transpose(2, 0, 1)

    CB = 4
    out_cbp = pl.pallas_call(
        _retrieve_kernel,
        grid=(C // CB,),
        in_specs=[
            pl.BlockSpec((CB, B, T), lambda i: (i, 0, 0)),
            pl.BlockSpec((CB, T, P), lambda i: (i, 0, 0)),
        ],
        out_specs=pl.BlockSpec((CB, B, P), lambda i: (i, 0, 0)),
        out_shape=jax.ShapeDtypeStruct((C, B, P), jnp.float32),
    )(sim_t, y_t)
    return out_cbp.transpose(1, 2, 0)
